# baseline (speedup 1.0000x reference)
import functools

import jax
import jax.numpy as jnp
from jax import lax
from jax.experimental import pallas as pl
from jax.experimental.pallas import tpu as pltpu
from jax.experimental.pallas import tpu_sc as plsc

N = 1000000
C = 64
B = 16384

_info = plsc.get_sparse_core_info()
_NC, _NS = _info.num_cores, _info.num_subcores
_NW = _NC * _NS
_B_PER_W = B // _NW

_mesh = plsc.VectorSubcoreMesh(core_axis_name="c", subcore_axis_name="s")


@functools.partial(
    pl.kernel,
    mesh=_mesh,
    out_type=jax.ShapeDtypeStruct((B, C), jnp.float32),
    compiler_params=pltpu.CompilerParams(use_tc_tiling_on_sc=False),
    scratch_types=[
        pltpu.VMEM((_B_PER_W,), jnp.int32),
        pltpu.VMEM((_B_PER_W, C), jnp.float32),
        pltpu.SemaphoreType.DMA,
    ],
)
def _gather_kernel(bank_hbm, idx_hbm, out_hbm, idx_v, rows_v, sem):
    wid = lax.axis_index("s") * _NC + lax.axis_index("c")
    base = wid * _B_PER_W
    pltpu.sync_copy(idx_hbm.at[pl.ds(base, _B_PER_W)], idx_v)
    pltpu.async_copy(bank_hbm.at[idx_v], rows_v, sem).wait()
    pltpu.sync_copy(rows_v, out_hbm.at[pl.ds(base, _B_PER_W)])


def kernel(logitsbank, index):
    return _gather_kernel(logitsbank, index)


# speedup vs baseline: 1.0350x; 1.0350x over previous
import functools

import jax
import jax.numpy as jnp
from jax import lax
from jax.experimental import pallas as pl
from jax.experimental.pallas import tpu as pltpu
from jax.experimental.pallas import tpu_sc as plsc

N = 1000000
C = 64
B = 16384

_info = plsc.get_sparse_core_info()
_NC, _NS = _info.num_cores, _info.num_subcores
_NW = _NC * _NS
_B_PER_W = B // _NW
_Q = 8

_mesh = plsc.VectorSubcoreMesh(core_axis_name="c", subcore_axis_name="s")


@functools.partial(
    pl.kernel,
    mesh=_mesh,
    out_type=jax.ShapeDtypeStruct((B, C), jnp.float32),
    compiler_params=pltpu.CompilerParams(needs_layout_passes=False),
    scratch_types=[
        pltpu.VMEM((_B_PER_W,), jnp.int32),
        pltpu.SemaphoreType.DMA,
    ],
)
def _gather_kernel(bank_hbm, idx_hbm, out_hbm, idx_v, sem):
    wid = lax.axis_index("s") * _NC + lax.axis_index("c")
    base = wid * _B_PER_W
    pltpu.sync_copy(idx_hbm.at[pl.ds(base, _B_PER_W)], idx_v)

    lanes = lax.iota(jnp.int32, 16)

    def fire(g):
        idxs = plsc.load_gather(idx_v, [g * 16 + lanes])
        for k in range(16):
            r = idxs[k]
            pltpu.async_copy(
                bank_hbm.at[pl.ds(r, 1)],
                out_hbm.at[pl.ds(base + g * 16 + k, 1)],
                sem,
            )

    def drain(g):
        pltpu.make_async_copy(
            bank_hbm.at[pl.ds(0, 16)],
            out_hbm.at[pl.ds(base + g * 16, 16)],
            sem,
        ).wait()

    fire(0)

    def body(g, _):
        fire(g)
        drain(g - 1)
        return 0

    lax.fori_loop(1, _B_PER_W // 16, body, 0)
    drain(_B_PER_W // 16 - 1)


def kernel(logitsbank, index):
    return _gather_kernel(logitsbank, index)


# speedup vs baseline: 1.7358x; 1.6770x over previous
import functools

import jax
import jax.numpy as jnp
from jax import lax
from jax.experimental import pallas as pl
from jax.experimental.pallas import tpu as pltpu
from jax.experimental.pallas import tpu_sc as plsc

N = 1000000
C = 64
B = 16384

_info = plsc.get_sparse_core_info()
_NC, _NS = _info.num_cores, _info.num_subcores
_NW = _NC * _NS
_B_PER_W = B // _NW

_mesh = plsc.VectorSubcoreMesh(core_axis_name="c", subcore_axis_name="s")


@functools.partial(
    pl.kernel,
    mesh=_mesh,
    out_type=jax.ShapeDtypeStruct((B, C), jnp.float32),
    compiler_params=pltpu.CompilerParams(needs_layout_passes=False),
    scratch_types=[
        pltpu.VMEM((_B_PER_W,), jnp.int32),
        pltpu.VMEM((_B_PER_W, C), jnp.float32),
        pltpu.SemaphoreType.DMA,
    ],
)
def _gather_kernel(bank_hbm, idx_hbm, out_hbm, idx_v, rows_v, sem):
    wid = lax.axis_index("s") * _NC + lax.axis_index("c")
    base = wid * _B_PER_W
    pltpu.sync_copy(idx_hbm.at[pl.ds(base, _B_PER_W)], idx_v)

    lanes = lax.iota(jnp.int32, 16)

    def fire_group(g, _):
        idxs = plsc.load_gather(idx_v, [g * 16 + lanes])
        for k in range(16):
            r = idxs[k]
            pltpu.async_copy(
                bank_hbm.at[pl.ds(r, 1)],
                rows_v.at[pl.ds(g * 16 + k, 1)],
                sem,
            )
        return 0

    lax.fori_loop(0, _B_PER_W // 16, fire_group, 0)
    pltpu.make_async_copy(bank_hbm.at[pl.ds(0, _B_PER_W)], rows_v, sem).wait()
    pltpu.sync_copy(rows_v, out_hbm.at[pl.ds(base, _B_PER_W)])


def kernel(logitsbank, index):
    return _gather_kernel(logitsbank, index)
